# baseline (speedup 1.0000x reference)
import functools

import jax
import jax.numpy as jnp
from jax import lax
from jax.experimental import pallas as pl
from jax.experimental.pallas import tpu as pltpu
from jax.experimental.pallas import tpu_sc as plsc

NH = 4
DH = 32
NCORES = 2
NTILES = 16

BN = 5000
CA = 1000
CB = 1600
RB = 320
NSB = CB // RB
ZA = 400
ZB = 200

_SC_PARAMS = pltpu.CompilerParams(
    use_tc_tiling_on_sc=False, needs_layout_passes=False)


def _mm_body(x_ref, w_ref, hh_ref, elr_ref):
    acc = jnp.dot(x_ref[...], w_ref[...], preferred_element_type=jnp.float32)
    for h in range(NH):
        hh_ref[h] = acc[:, h * DH:(h + 1) * DH]
    elr_ref[...] = acc[:, NH * DH:NH * DH + 8]


def _mm_call(x, wext, n):
    return pl.pallas_call(
        _mm_body,
        grid=(n // BN,),
        in_specs=[
            pl.BlockSpec((BN, x.shape[1]), lambda i: (i, 0)),
            pl.BlockSpec(wext.shape, lambda i: (0, 0)),
        ],
        out_specs=[
            pl.BlockSpec((NH, BN, DH), lambda i: (0, i, 0)),
            pl.BlockSpec((BN, 8), lambda i: (i, 0)),
        ],
        out_shape=[
            jax.ShapeDtypeStruct((NH, n, DH), jnp.float32),
            jax.ShapeDtypeStruct((n, 8), jnp.float32),
        ],
    )(x, wext)


def _zero_rows8(ref, nrows, col_lo, ncols):
    lane = lax.iota(jnp.int32, 16)
    rpv = 16 // ncols
    ridx0 = lane // ncols
    cidx = lane % ncols + col_lo
    zero = jnp.zeros((16,), jnp.float32)

    def _it(g, _):
        plsc.store_scatter(ref, [ridx0 + g * rpv, cidx], zero)
        return 0
    lax.fori_loop(0, nrows // rpv, _it, 0)


def _edge_w_body(elr_hbm, src_hbm, dst_hbm, w_hbm, den_hbm,
                 elr_sh, den_sh, src_v, dst_v, srcrows, dstrows, w4, w_t, stage,
                 asem):
    c = lax.axis_index("c")
    s = lax.axis_index("s")
    wid = c * NTILES + s
    n = elr_hbm.shape[0]
    e_total = src_hbm.shape[0]
    nz = n // ZA
    nj = nz // NTILES + jnp.where(s < nz % NTILES, 1, 0)

    _zero_rows8(w4, CA, 4, 4)
    _zero_rows8(stage, ZA, 0, 8)

    def _zden(j, _):
        pltpu.sync_copy(stage, den_sh.at[pl.ds((s + j * NTILES) * ZA, ZA)])
        return 0
    lax.fori_loop(0, nj, _zden, 0)

    def _stg(j, _):
        lo = (s + j * NTILES) * ZA
        pltpu.sync_copy(elr_hbm.at[pl.ds(lo, ZA)], stage)
        pltpu.sync_copy(stage, elr_sh.at[pl.ds(lo, ZA)])
        return 0
    lax.fori_loop(0, nj, _stg, 0)
    plsc.subcore_barrier()

    ept = e_total // (NCORES * NTILES)
    base = wid * ept
    lane = lax.iota(jnp.int32, 16)
    ridx0 = lane // 4
    cidx = lane % 4

    def _chunk(t, _):
        lo = base + t * CA
        h1 = pltpu.async_copy(src_hbm.at[pl.ds(lo, CA)], src_v, asem)
        h2 = pltpu.async_copy(dst_hbm.at[pl.ds(lo, CA)], dst_v, asem)
        h1.wait()
        h2.wait()
        h3 = pltpu.async_copy(elr_sh.at[src_v], srcrows, asem)
        h4 = pltpu.async_copy(elr_sh.at[dst_v], dstrows, asem)
        h3.wait()
        h4.wait()

        @plsc.parallel_loop(0, CA // 4, 1, unroll=2)
        def _grp(g):
            ridx = ridx0 + g * 4
            a = plsc.load_gather(srcrows, [ridx, cidx])
            b = plsc.load_gather(dstrows, [ridx, cidx + 4])
            t_ = a + b
            t_ = jnp.where(t_ >= 0, t_, 0.2 * t_)
            wv = jnp.exp(t_)
            plsc.store_scatter(w4, [ridx, cidx], wv)
            plsc.store_scatter(w_t, [cidx * CA + ridx], wv)

        pltpu.sync_copy(w4, den_sh.at[dst_v], add=True)
        for h in range(NH):
            pltpu.sync_copy(w_t.at[pl.ds(h * CA, CA)],
                            w_hbm.at[pl.ds(h * e_total + lo, CA)])
        return 0
    lax.fori_loop(0, ept // CA, _chunk, 0)
    plsc.subcore_barrier()

    def _flush(j, _):
        lo = (s + j * NTILES) * ZA
        pltpu.sync_copy(den_sh.at[pl.ds(lo, ZA)], stage)
        pltpu.sync_copy(stage, den_hbm.at[c, pl.ds(lo, ZA)])
        return 0
    lax.fori_loop(0, nj, _flush, 0)


def _edge_w_call(elr, src1d, dst1d, n, e):
    mesh = plsc.VectorSubcoreMesh(core_axis_name="c", subcore_axis_name="s")
    k = pl.kernel(
        _edge_w_body,
        out_type=[
            jax.ShapeDtypeStruct((NH * e,), jnp.float32),
            jax.ShapeDtypeStruct((NCORES, n, 8), jnp.float32),
        ],
        mesh=mesh,
        scratch_types=[
            pltpu.VMEM_SHARED((n, 8), jnp.float32),
            pltpu.VMEM_SHARED((n, 8), jnp.float32),
            pltpu.VMEM((CA,), jnp.int32),
            pltpu.VMEM((CA,), jnp.int32),
            pltpu.VMEM((CA, 8), jnp.float32),
            pltpu.VMEM((CA, 8), jnp.float32),
            pltpu.VMEM((CA, 8), jnp.float32),
            pltpu.VMEM((NH * CA,), jnp.float32),
            pltpu.VMEM((ZA, 8), jnp.float32),
            pltpu.SemaphoreType.DMA,
        ],
        compiler_params=_SC_PARAMS,
    )
    return k(elr, src1d, dst1d)


def _msg_body(hh_hbm, src2_hbm, dst2_hbm, w_hbm, out_hbm,
              acc_sh, idx_v, dst_v, wv_v, rows0, rows1, gsem, ssem):
    c = lax.axis_index("c")
    s = lax.axis_index("s")
    n = out_hbm.shape[1]
    e_total = src2_hbm.shape[0] * src2_hbm.shape[1]
    nzb = n // ZB
    njz = nzb // NTILES + jnp.where(s < nzb % NTILES, 1, 0)
    nch = e_total // CB
    njc = nch // NTILES + jnp.where(s < nch % NTILES, 1, 0)
    bufs = (rows0, rows1)

    for r in range(2):
        head = c * 2 + r
        off = head * n

        def _zr(i, _):
            rows0[i, 0:16] = jnp.zeros((16,), jnp.float32)
            rows0[i, 16:32] = jnp.zeros((16,), jnp.float32)
            return 0
        lax.fori_loop(0, ZB, _zr, 0)

        def _za(j, _):
            pltpu.sync_copy(rows0.at[pl.ds(0, ZB)],
                            acc_sh.at[pl.ds((s + j * NTILES) * ZB, ZB)])
            return 0
        lax.fori_loop(0, njz, _za, 0)
        plsc.subcore_barrier()

        def _chunk(j, _):
            k = s + j * NTILES
            lo = k * CB
            h1 = pltpu.async_copy(
                w_hbm.at[pl.ds(head * e_total + lo, CB)], wv_v, gsem)
            h2 = pltpu.async_copy(src2_hbm.at[pl.ds(k * NSB, NSB)], idx_v, gsem)
            h3 = pltpu.async_copy(dst2_hbm.at[pl.ds(k * NSB, NSB)], dst_v, gsem)
            h1.wait()
            h2.wait()
            h3.wait()

            @plsc.parallel_loop(0, CB // 16, 1, unroll=4)
            def _ix(g):
                sb2 = g // (RB // 16)
                g2 = g % (RB // 16)
                sl = pl.ds(g2 * 16, 16)
                idx_v[sb2, sl] = idx_v[sb2, sl] + off

            gh = [None] * NSB
            sh = [None] * NSB
            gh[0] = pltpu.async_copy(hh_hbm.at[idx_v.at[0]], bufs[0], gsem)
            for sb in range(NSB):
                cur = bufs[sb % 2]
                gh[sb].wait()
                if sb + 1 < NSB:
                    if sb >= 1:
                        sh[sb - 1].wait()
                    gh[sb + 1] = pltpu.async_copy(
                        hh_hbm.at[idx_v.at[sb + 1]], bufs[(sb + 1) % 2], gsem)

                @plsc.parallel_loop(0, RB // 16, 1, unroll=4)
                def _sc(g):
                    wvec = wv_v[pl.ds(sb * RB + g * 16, 16)]
                    for jj in range(16):
                        i = g * 16 + jj
                        wsc = wvec[jj]
                        cur[i, 0:16] = cur[i, 0:16] * wsc
                        cur[i, 16:32] = cur[i, 16:32] * wsc
                sh[sb] = pltpu.async_copy(cur, acc_sh.at[dst_v.at[sb]], ssem,
                                          add=True)
            sh[NSB - 2].wait()
            sh[NSB - 1].wait()
            return 0
        lax.fori_loop(0, njc, _chunk, 0)
        plsc.subcore_barrier()

        def _fl(j, _):
            lo2 = (s + j * NTILES) * ZB
            pltpu.sync_copy(acc_sh.at[pl.ds(lo2, ZB)],
                            out_hbm.at[head, pl.ds(lo2, ZB)])
            return 0
        lax.fori_loop(0, njz, _fl, 0)
        plsc.subcore_barrier()


def _msg_call(hh_flat, src2d, dst2d, w_pl, n):
    mesh = plsc.VectorSubcoreMesh(core_axis_name="c", subcore_axis_name="s")
    k = pl.kernel(
        _msg_body,
        out_type=jax.ShapeDtypeStruct((NH, n, DH), jnp.float32),
        mesh=mesh,
        scratch_types=[
            pltpu.VMEM_SHARED((n, DH), jnp.float32),
            pltpu.VMEM((NSB, RB), jnp.int32),
            pltpu.VMEM((NSB, RB), jnp.int32),
            pltpu.VMEM((CB,), jnp.float32),
            pltpu.VMEM((RB, DH), jnp.float32),
            pltpu.VMEM((RB, DH), jnp.float32),
            pltpu.SemaphoreType.DMA,
            pltpu.SemaphoreType.DMA,
        ],
        compiler_params=_SC_PARAMS,
    )
    return k(hh_flat, src2d, dst2d, w_pl)


def _c0_body(ou_ref, den_ref, gb_ref, z_ref, st_ref):
    den = den_ref[0] + den_ref[1]
    parts = []
    for h in range(NH):
        d = den[:, h][:, None] + 1e-16
        parts.append(ou_ref[h] / d)
    z = jnp.concatenate(parts, axis=1) + gb_ref[...]
    z_ref[...] = z
    st = jnp.concatenate([jnp.sum(z, 0)[None], jnp.sum(z * z, 0)[None]], 0)

    @pl.when(pl.program_id(0) == 0)
    def _():
        st_ref[...] = jnp.zeros_like(st_ref)
    st_ref[...] += st


def _c0_call(out_un, den, gb, n):
    d = NH * DH
    return pl.pallas_call(
        _c0_body,
        grid=(n // BN,),
        in_specs=[
            pl.BlockSpec((NH, BN, DH), lambda i: (0, i, 0)),
            pl.BlockSpec((NCORES, BN, 8), lambda i: (0, i, 0)),
            pl.BlockSpec((1, d), lambda i: (0, 0)),
        ],
        out_specs=[
            pl.BlockSpec((BN, d), lambda i: (i, 0)),
            pl.BlockSpec((2, d), lambda i: (0, 0)),
        ],
        out_shape=[
            jax.ShapeDtypeStruct((n, d), jnp.float32),
            jax.ShapeDtypeStruct((2, d), jnp.float32),
        ],
    )(out_un, den, gb)


def _layer_body(nrows, slope, z_ref, st_ref, g_ref, b_ref, w_ref, bias_ref,
                out_ref, stn_ref):
    mu = st_ref[0:1, :] / nrows
    var = st_ref[1:2, :] / nrows - mu * mu
    sc_ = g_ref[...] / jnp.sqrt(var + 1e-5)
    sh_ = b_ref[...] - mu * sc_
    a = z_ref[...] * sc_ + sh_
    a = jnp.where(a >= 0, a, slope * a)
    y = jnp.dot(a, w_ref[...], preferred_element_type=jnp.float32) + bias_ref[...]
    out_ref[...] = y
    st = jnp.concatenate([jnp.sum(y, 0)[None], jnp.sum(y * y, 0)[None]], 0)

    @pl.when(pl.program_id(0) == 0)
    def _():
        stn_ref[...] = jnp.zeros_like(stn_ref)
    stn_ref[...] += st


def _layer_call(z, st, g, b, w, bias, n):
    dk = z.shape[1]
    dn = w.shape[1]
    return pl.pallas_call(
        functools.partial(_layer_body, float(n), 0.1),
        grid=(n // BN,),
        in_specs=[
            pl.BlockSpec((BN, dk), lambda i: (i, 0)),
            pl.BlockSpec((2, dk), lambda i: (0, 0)),
            pl.BlockSpec((1, dk), lambda i: (0, 0)),
            pl.BlockSpec((1, dk), lambda i: (0, 0)),
            pl.BlockSpec((dk, dn), lambda i: (0, 0)),
            pl.BlockSpec((1, dn), lambda i: (0, 0)),
        ],
        out_specs=[
            pl.BlockSpec((BN, dn), lambda i: (i, 0)),
            pl.BlockSpec((2, dn), lambda i: (0, 0)),
        ],
        out_shape=[
            jax.ShapeDtypeStruct((n, dn), jnp.float32),
            jax.ShapeDtypeStruct((2, dn), jnp.float32),
        ],
    )(z, st, g, b, w, bias)


def _c5_body(nrows, z_ref, st_ref, g_ref, b_ref, out_ref):
    mu = st_ref[0:1, :] / nrows
    var = st_ref[1:2, :] / nrows - mu * mu
    sc_ = g_ref[...] / jnp.sqrt(var + 1e-5)
    a = z_ref[...] * sc_ + (b_ref[...] - mu * sc_)
    a = jnp.where(a >= 0, a, 0.1 * a)
    m = jnp.max(a, axis=1, keepdims=True)
    ex = jnp.exp(a - m)
    out_ref[...] = ex / jnp.sum(ex, axis=1, keepdims=True)


def _c5_call(z, st, g, b, n):
    dk = z.shape[1]
    return pl.pallas_call(
        functools.partial(_c5_body, float(n)),
        grid=(n // BN,),
        in_specs=[
            pl.BlockSpec((BN, dk), lambda i: (i, 0)),
            pl.BlockSpec((2, dk), lambda i: (0, 0)),
            pl.BlockSpec((1, dk), lambda i: (0, 0)),
            pl.BlockSpec((1, dk), lambda i: (0, 0)),
        ],
        out_specs=pl.BlockSpec((BN, dk), lambda i: (i, 0)),
        out_shape=jax.ShapeDtypeStruct((n, dk), jnp.float32),
    )(z, st, g, b)


def kernel(x, adj, W, a_l, a_r, gat_b, fc1_W, fc1_b, fc2_W, fc2_b, fc3_W,
           fc3_b, fc4_W, fc4_b, g1, b1, g2, b2, g3, b3, g4, b4, g5, b5):
    n, din = x.shape
    e = adj.shape[1]
    wr3 = W.reshape(din, NH, DH)
    wl = jnp.einsum('kho,ho->kh', wr3, a_l)
    wr = jnp.einsum('kho,ho->kh', wr3, a_r)
    wext = jnp.concatenate([W, wl, wr], axis=1)

    hh, elr = _mm_call(x, wext, n)
    src1d = adj[0]
    dst1d = adj[1]
    w_pl, den = _edge_w_call(elr, src1d, dst1d, n, e)
    out_un = _msg_call(hh.reshape(NH * n, DH),
                       src1d.reshape(e // RB, RB),
                       dst1d.reshape(e // RB, RB), w_pl, n)

    z, st = _c0_call(out_un, den, gat_b.reshape(1, -1), n)
    z, st = _layer_call(z, st, g1.reshape(1, -1), b1.reshape(1, -1),
                        fc1_W, fc1_b.reshape(1, -1), n)
    z, st = _layer_call(z, st, g2.reshape(1, -1), b2.reshape(1, -1),
                        fc2_W, fc2_b.reshape(1, -1), n)
    z, st = _layer_call(z, st, g3.reshape(1, -1), b3.reshape(1, -1),
                        fc3_W, fc3_b.reshape(1, -1), n)
    z, st = _layer_call(z, st, g4.reshape(1, -1), b4.reshape(1, -1),
                        fc4_W, fc4_b.reshape(1, -1), n)
    return _c5_call(z, st, g5.reshape(1, -1), b5.reshape(1, -1), n)


# speedup vs baseline: 1.0362x; 1.0362x over previous
import functools

import jax
import jax.numpy as jnp
from jax import lax
from jax.experimental import pallas as pl
from jax.experimental.pallas import tpu as pltpu
from jax.experimental.pallas import tpu_sc as plsc

NH = 4
DH = 32
NCORES = 2
NTILES = 16

BN = 5000
CA = 1000
CB = 1600
RB = 320
NSB = CB // RB
ZA = 400
ZB = 200

_SC_PARAMS = pltpu.CompilerParams(
    use_tc_tiling_on_sc=False, needs_layout_passes=False)


def _mm_body(x_ref, w_ref, hh_ref, elr_ref):
    acc = jnp.dot(x_ref[...], w_ref[...], preferred_element_type=jnp.float32)
    for h in range(NH):
        hh_ref[h] = acc[:, h * DH:(h + 1) * DH]
    elr_ref[...] = acc[:, NH * DH:NH * DH + 8]


def _mm_call(x, wext, n):
    return pl.pallas_call(
        _mm_body,
        grid=(n // BN,),
        in_specs=[
            pl.BlockSpec((BN, x.shape[1]), lambda i: (i, 0)),
            pl.BlockSpec(wext.shape, lambda i: (0, 0)),
        ],
        out_specs=[
            pl.BlockSpec((NH, BN, DH), lambda i: (0, i, 0)),
            pl.BlockSpec((BN, 8), lambda i: (i, 0)),
        ],
        out_shape=[
            jax.ShapeDtypeStruct((NH, n, DH), jnp.float32),
            jax.ShapeDtypeStruct((n, 8), jnp.float32),
        ],
    )(x, wext)


def _zero_rows8(ref, nrows, col_lo, ncols):
    lane = lax.iota(jnp.int32, 16)
    rpv = 16 // ncols
    ridx0 = lane // ncols
    cidx = lane % ncols + col_lo
    zero = jnp.zeros((16,), jnp.float32)

    def _it(g, _):
        plsc.store_scatter(ref, [ridx0 + g * rpv, cidx], zero)
        return 0
    lax.fori_loop(0, nrows // rpv, _it, 0)


def _edge_w_body(elr_hbm, src_hbm, dst_hbm, w_hbm, den_hbm,
                 elr_sh, den_sh, src_v, dst_v, srcrows, dstrows, w4, w_t, stage,
                 asem):
    c = lax.axis_index("c")
    s = lax.axis_index("s")
    wid = c * NTILES + s
    n = elr_hbm.shape[0]
    e_total = src_hbm.shape[0]
    nz = n // ZA
    nj = nz // NTILES + jnp.where(s < nz % NTILES, 1, 0)

    _zero_rows8(w4, CA, 4, 4)
    _zero_rows8(stage, ZA, 0, 8)

    def _zden(j, _):
        pltpu.sync_copy(stage, den_sh.at[pl.ds((s + j * NTILES) * ZA, ZA)])
        return 0
    lax.fori_loop(0, nj, _zden, 0)

    def _stg(j, _):
        lo = (s + j * NTILES) * ZA
        pltpu.sync_copy(elr_hbm.at[pl.ds(lo, ZA)], stage)
        pltpu.sync_copy(stage, elr_sh.at[pl.ds(lo, ZA)])
        return 0
    lax.fori_loop(0, nj, _stg, 0)
    plsc.subcore_barrier()

    ept = e_total // (NCORES * NTILES)
    base = wid * ept
    lane = lax.iota(jnp.int32, 16)
    ridx0 = lane // 4
    cidx = lane % 4

    def _chunk(t, _):
        lo = base + t * CA
        h1 = pltpu.async_copy(src_hbm.at[pl.ds(lo, CA)], src_v, asem)
        h2 = pltpu.async_copy(dst_hbm.at[pl.ds(lo, CA)], dst_v, asem)
        h1.wait()
        h2.wait()
        h3 = pltpu.async_copy(elr_sh.at[src_v], srcrows, asem)
        h4 = pltpu.async_copy(elr_sh.at[dst_v], dstrows, asem)
        h3.wait()
        h4.wait()

        @plsc.parallel_loop(0, CA // 4, 1, unroll=2)
        def _grp(g):
            ridx = ridx0 + g * 4
            a = plsc.load_gather(srcrows, [ridx, cidx])
            b = plsc.load_gather(dstrows, [ridx, cidx + 4])
            t_ = a + b
            t_ = jnp.where(t_ >= 0, t_, 0.2 * t_)
            wv = jnp.exp(t_)
            plsc.store_scatter(w4, [ridx, cidx], wv)
            plsc.store_scatter(w_t, [cidx * CA + ridx], wv)

        pltpu.sync_copy(w4, den_sh.at[dst_v], add=True)
        for h in range(NH):
            pltpu.sync_copy(w_t.at[pl.ds(h * CA, CA)],
                            w_hbm.at[pl.ds(h * e_total + lo, CA)])
        return 0
    lax.fori_loop(0, ept // CA, _chunk, 0)
    plsc.subcore_barrier()

    def _flush(j, _):
        lo = (s + j * NTILES) * ZA
        pltpu.sync_copy(den_sh.at[pl.ds(lo, ZA)], stage)
        pltpu.sync_copy(stage, den_hbm.at[c, pl.ds(lo, ZA)])
        return 0
    lax.fori_loop(0, nj, _flush, 0)


def _edge_w_call(elr, src1d, dst1d, n, e):
    mesh = plsc.VectorSubcoreMesh(core_axis_name="c", subcore_axis_name="s")
    k = pl.kernel(
        _edge_w_body,
        out_type=[
            jax.ShapeDtypeStruct((NH * e,), jnp.float32),
            jax.ShapeDtypeStruct((NCORES, n, 8), jnp.float32),
        ],
        mesh=mesh,
        scratch_types=[
            pltpu.VMEM_SHARED((n, 8), jnp.float32),
            pltpu.VMEM_SHARED((n, 8), jnp.float32),
            pltpu.VMEM((CA,), jnp.int32),
            pltpu.VMEM((CA,), jnp.int32),
            pltpu.VMEM((CA, 8), jnp.float32),
            pltpu.VMEM((CA, 8), jnp.float32),
            pltpu.VMEM((CA, 8), jnp.float32),
            pltpu.VMEM((NH * CA,), jnp.float32),
            pltpu.VMEM((ZA, 8), jnp.float32),
            pltpu.SemaphoreType.DMA,
        ],
        compiler_params=_SC_PARAMS,
    )
    return k(elr, src1d, dst1d)


def _msg_body(hh_hbm, src2_hbm, dst2_hbm, w_hbm, out_hbm,
              acc_sh, idx_v, dst_v, wv_v, rows0, rows1, gsem, ssem):
    c = lax.axis_index("c")
    s = lax.axis_index("s")
    n = out_hbm.shape[1]
    e_total = src2_hbm.shape[0] * src2_hbm.shape[1]
    nzb = n // ZB
    njz = nzb // NTILES + jnp.where(s < nzb % NTILES, 1, 0)
    nch = e_total // CB
    njc = nch // NTILES + jnp.where(s < nch % NTILES, 1, 0)
    bufs = (rows0, rows1)

    for r in range(2):
        head = c * 2 + r
        off = head * n

        def _zr(i, _):
            rows0[i, 0:16] = jnp.zeros((16,), jnp.float32)
            rows0[i, 16:32] = jnp.zeros((16,), jnp.float32)
            return 0
        lax.fori_loop(0, ZB, _zr, 0)

        def _za(j, _):
            pltpu.sync_copy(rows0.at[pl.ds(0, ZB)],
                            acc_sh.at[pl.ds((s + j * NTILES) * ZB, ZB)])
            return 0
        lax.fori_loop(0, njz, _za, 0)
        plsc.subcore_barrier()

        def _chunk(j, _):
            k = s + j * NTILES
            lo = k * CB
            h1 = pltpu.async_copy(
                w_hbm.at[pl.ds(head * e_total + lo, CB)], wv_v, gsem)
            h2 = pltpu.async_copy(src2_hbm.at[pl.ds(k * NSB, NSB)], idx_v, gsem)
            h3 = pltpu.async_copy(dst2_hbm.at[pl.ds(k * NSB, NSB)], dst_v, gsem)
            h1.wait()
            h2.wait()
            h3.wait()

            @plsc.parallel_loop(0, CB // 16, 1, unroll=2)
            def _ix(g):
                sb2 = g // (RB // 16)
                g2 = g % (RB // 16)
                sl = pl.ds(g2 * 16, 16)
                idx_v[sb2, sl] = idx_v[sb2, sl] + off

            gh = [None] * NSB
            sh = [None] * NSB
            gh[0] = pltpu.async_copy(hh_hbm.at[idx_v.at[0]], bufs[0], gsem)
            for sb in range(NSB):
                cur = bufs[sb % 2]
                gh[sb].wait()
                if sb + 1 < NSB:
                    if sb >= 1:
                        sh[sb - 1].wait()
                    gh[sb + 1] = pltpu.async_copy(
                        hh_hbm.at[idx_v.at[sb + 1]], bufs[(sb + 1) % 2], gsem)

                @plsc.parallel_loop(0, RB // 16, 1, unroll=2)
                def _sc(g):
                    wvec = wv_v[pl.ds(sb * RB + g * 16, 16)]
                    for jj in range(16):
                        i = g * 16 + jj
                        wsc = wvec[jj]
                        cur[i, 0:16] = cur[i, 0:16] * wsc
                        cur[i, 16:32] = cur[i, 16:32] * wsc
                sh[sb] = pltpu.async_copy(cur, acc_sh.at[dst_v.at[sb]], ssem,
                                          add=True)
            sh[NSB - 2].wait()
            sh[NSB - 1].wait()
            return 0
        lax.fori_loop(0, njc, _chunk, 0)
        plsc.subcore_barrier()

        def _fl(j, _):
            lo2 = (s + j * NTILES) * ZB
            pltpu.sync_copy(acc_sh.at[pl.ds(lo2, ZB)],
                            out_hbm.at[head, pl.ds(lo2, ZB)])
            return 0
        lax.fori_loop(0, njz, _fl, 0)
        plsc.subcore_barrier()


def _msg_call(hh_flat, src2d, dst2d, w_pl, n):
    mesh = plsc.VectorSubcoreMesh(core_axis_name="c", subcore_axis_name="s")
    k = pl.kernel(
        _msg_body,
        out_type=jax.ShapeDtypeStruct((NH, n, DH), jnp.float32),
        mesh=mesh,
        scratch_types=[
            pltpu.VMEM_SHARED((n, DH), jnp.float32),
            pltpu.VMEM((NSB, RB), jnp.int32),
            pltpu.VMEM((NSB, RB), jnp.int32),
            pltpu.VMEM((CB,), jnp.float32),
            pltpu.VMEM((RB, DH), jnp.float32),
            pltpu.VMEM((RB, DH), jnp.float32),
            pltpu.SemaphoreType.DMA,
            pltpu.SemaphoreType.DMA,
        ],
        compiler_params=_SC_PARAMS,
    )
    return k(hh_flat, src2d, dst2d, w_pl)


def _c0_body(ou_ref, den_ref, gb_ref, z_ref, st_ref):
    den = den_ref[0] + den_ref[1]
    parts = []
    for h in range(NH):
        d = den[:, h][:, None] + 1e-16
        parts.append(ou_ref[h] / d)
    z = jnp.concatenate(parts, axis=1) + gb_ref[...]
    z_ref[...] = z
    st = jnp.concatenate([jnp.sum(z, 0)[None], jnp.sum(z * z, 0)[None]], 0)

    @pl.when(pl.program_id(0) == 0)
    def _():
        st_ref[...] = jnp.zeros_like(st_ref)
    st_ref[...] += st


def _c0_call(out_un, den, gb, n):
    d = NH * DH
    return pl.pallas_call(
        _c0_body,
        grid=(n // BN,),
        in_specs=[
            pl.BlockSpec((NH, BN, DH), lambda i: (0, i, 0)),
            pl.BlockSpec((NCORES, BN, 8), lambda i: (0, i, 0)),
            pl.BlockSpec((1, d), lambda i: (0, 0)),
        ],
        out_specs=[
            pl.BlockSpec((BN, d), lambda i: (i, 0)),
            pl.BlockSpec((2, d), lambda i: (0, 0)),
        ],
        out_shape=[
            jax.ShapeDtypeStruct((n, d), jnp.float32),
            jax.ShapeDtypeStruct((2, d), jnp.float32),
        ],
    )(out_un, den, gb)


def _layer_body(nrows, slope, z_ref, st_ref, g_ref, b_ref, w_ref, bias_ref,
                out_ref, stn_ref):
    mu = st_ref[0:1, :] / nrows
    var = st_ref[1:2, :] / nrows - mu * mu
    sc_ = g_ref[...] / jnp.sqrt(var + 1e-5)
    sh_ = b_ref[...] - mu * sc_
    a = z_ref[...] * sc_ + sh_
    a = jnp.where(a >= 0, a, slope * a)
    y = jnp.dot(a, w_ref[...], preferred_element_type=jnp.float32) + bias_ref[...]
    out_ref[...] = y
    st = jnp.concatenate([jnp.sum(y, 0)[None], jnp.sum(y * y, 0)[None]], 0)

    @pl.when(pl.program_id(0) == 0)
    def _():
        stn_ref[...] = jnp.zeros_like(stn_ref)
    stn_ref[...] += st


def _layer_call(z, st, g, b, w, bias, n):
    dk = z.shape[1]
    dn = w.shape[1]
    return pl.pallas_call(
        functools.partial(_layer_body, float(n), 0.1),
        grid=(n // BN,),
        in_specs=[
            pl.BlockSpec((BN, dk), lambda i: (i, 0)),
            pl.BlockSpec((2, dk), lambda i: (0, 0)),
            pl.BlockSpec((1, dk), lambda i: (0, 0)),
            pl.BlockSpec((1, dk), lambda i: (0, 0)),
            pl.BlockSpec((dk, dn), lambda i: (0, 0)),
            pl.BlockSpec((1, dn), lambda i: (0, 0)),
        ],
        out_specs=[
            pl.BlockSpec((BN, dn), lambda i: (i, 0)),
            pl.BlockSpec((2, dn), lambda i: (0, 0)),
        ],
        out_shape=[
            jax.ShapeDtypeStruct((n, dn), jnp.float32),
            jax.ShapeDtypeStruct((2, dn), jnp.float32),
        ],
    )(z, st, g, b, w, bias)


def _c5_body(nrows, z_ref, st_ref, g_ref, b_ref, out_ref):
    mu = st_ref[0:1, :] / nrows
    var = st_ref[1:2, :] / nrows - mu * mu
    sc_ = g_ref[...] / jnp.sqrt(var + 1e-5)
    a = z_ref[...] * sc_ + (b_ref[...] - mu * sc_)
    a = jnp.where(a >= 0, a, 0.1 * a)
    m = jnp.max(a, axis=1, keepdims=True)
    ex = jnp.exp(a - m)
    out_ref[...] = ex / jnp.sum(ex, axis=1, keepdims=True)


def _c5_call(z, st, g, b, n):
    dk = z.shape[1]
    return pl.pallas_call(
        functools.partial(_c5_body, float(n)),
        grid=(n // BN,),
        in_specs=[
            pl.BlockSpec((BN, dk), lambda i: (i, 0)),
            pl.BlockSpec((2, dk), lambda i: (0, 0)),
            pl.BlockSpec((1, dk), lambda i: (0, 0)),
            pl.BlockSpec((1, dk), lambda i: (0, 0)),
        ],
        out_specs=pl.BlockSpec((BN, dk), lambda i: (i, 0)),
        out_shape=jax.ShapeDtypeStruct((n, dk), jnp.float32),
    )(z, st, g, b)


def kernel(x, adj, W, a_l, a_r, gat_b, fc1_W, fc1_b, fc2_W, fc2_b, fc3_W,
           fc3_b, fc4_W, fc4_b, g1, b1, g2, b2, g3, b3, g4, b4, g5, b5):
    n, din = x.shape
    e = adj.shape[1]
    wr3 = W.reshape(din, NH, DH)
    wl = jnp.einsum('kho,ho->kh', wr3, a_l)
    wr = jnp.einsum('kho,ho->kh', wr3, a_r)
    wext = jnp.concatenate([W, wl, wr], axis=1)

    hh, elr = _mm_call(x, wext, n)
    src1d = adj[0]
    dst1d = adj[1]
    w_pl, den = _edge_w_call(elr, src1d, dst1d, n, e)
    out_un = _msg_call(hh.reshape(NH * n, DH),
                       src1d.reshape(e // RB, RB),
                       dst1d.reshape(e // RB, RB), w_pl, n)

    z, st = _c0_call(out_un, den, gat_b.reshape(1, -1), n)
    z, st = _layer_call(z, st, g1.reshape(1, -1), b1.reshape(1, -1),
                        fc1_W, fc1_b.reshape(1, -1), n)
    z, st = _layer_call(z, st, g2.reshape(1, -1), b2.reshape(1, -1),
                        fc2_W, fc2_b.reshape(1, -1), n)
    z, st = _layer_call(z, st, g3.reshape(1, -1), b3.reshape(1, -1),
                        fc3_W, fc3_b.reshape(1, -1), n)
    z, st = _layer_call(z, st, g4.reshape(1, -1), b4.reshape(1, -1),
                        fc4_W, fc4_b.reshape(1, -1), n)
    return _c5_call(z, st, g5.reshape(1, -1), b5.reshape(1, -1), n)
